# baseline (speedup 1.0000x reference)
import functools

import jax
import jax.numpy as jnp
from jax import lax
from jax.experimental import pallas as pl
from jax.experimental.pallas import tpu as pltpu
from jax.experimental.pallas import tpu_sc as plsc

_N = 10000
_E = 160000
_NP = 10240
_NR = 10112
_NC = 2
_NS = 16
_K = 64
_CH = 160
_EPT = _CH * _K
_EP = _NS * _EPT
_OPT = _NR // _NS
_PAD_ID = 10000


def _sc_agg_body(n_pass, with_counts, *refs):
    it = iter(refs)
    table = next(it)
    src_h = next(it)
    dst_h = next(it)
    zrow_h = next(it)
    if with_counts:
        zflat_h = next(it)
    agg_o = next(it)
    if with_counts:
        cnt_o = next(it)
    acc = next(it)
    idx_s = next(it)
    idx_d = next(it)
    rows = [next(it), next(it)]
    if with_counts:
        hist = next(it)
    gsem = [next(it), next(it)]
    ssem = [next(it), next(it)]

    c = lax.axis_index("c")
    t = lax.axis_index("s")

    pltpu.sync_copy(src_h.at[t], idx_s)
    pltpu.sync_copy(dst_h.at[t], idx_d)
    if with_counts:
        @pl.when(c == 0)
        def _():
            pltpu.sync_copy(zflat_h, hist)

    dummy = zrow_h.at[pl.ds(0, _K)]
    ones = jnp.full((16,), 1.0, jnp.float32)

    def count_chunk(k):
        @pl.when(c == 0)
        def _():
            for i in range(_K // 16):
                v = idx_d[pl.ds(k * _K + i * 16, 16)]
                plsc.addupdate_scatter(hist, [v], ones)

    for p in range(n_pass):
        pltpu.sync_copy(zrow_h, acc.at[pl.ds(t * _OPT, _OPT)])
        plsc.subcore_barrier()

        chunk = 2 * p + c
        count = count_chunk if (with_counts and p == 0) else (lambda k: None)

        def g_issue(k, b):
            pltpu.async_copy(table.at[chunk].at[idx_s.at[pl.ds(k * _K, _K)]],
                             rows[b], gsem[b])

        def g_wait(b):
            pltpu.make_async_copy(dummy, rows[b], gsem[b]).wait()

        def s_issue(k, b):
            pltpu.async_copy(rows[b], acc.at[idx_d.at[pl.ds(k * _K, _K)]],
                             ssem[b], add=True)

        def s_wait(b):
            pltpu.make_async_copy(dummy, rows[b], ssem[b]).wait()

        g_issue(0, 0)
        g_wait(0)
        s_issue(0, 0)
        count(0)
        g_issue(1, 1)

        def steady(i, carry):
            for (d, b) in ((1, 1), (2, 0)):
                k = 2 * i + d
                g_wait(b)
                s_issue(k, b)
                count(k)
                s_wait(1 - b)
                g_issue(k + 1, 1 - b)
            return carry

        lax.fori_loop(0, (_CH - 2) // 2, steady, 0)

        g_wait(1)
        s_issue(_CH - 1, 1)
        count(_CH - 1)
        s_wait(0)
        s_wait(1)

        plsc.subcore_barrier()

        pltpu.sync_copy(acc.at[pl.ds(t * _OPT, _OPT)],
                        agg_o.at[chunk].at[pl.ds(t * _OPT, _OPT)])
        plsc.subcore_barrier()

    if with_counts:
        @pl.when(c == 0)
        def _():
            pltpu.sync_copy(hist, cnt_o.at[t])


@functools.lru_cache(maxsize=None)
def _make_sc_agg(n_pass, with_counts):
    f32 = jnp.float32
    out_type = [jax.ShapeDtypeStruct((n_pass * 2, _NR, 128), f32)]
    if with_counts:
        out_type.append(jax.ShapeDtypeStruct((_NS, _NR), f32))
    scratch = [
        pltpu.VMEM_SHARED((_NR, 128), f32),
        pltpu.VMEM((_EPT,), jnp.int32),
        pltpu.VMEM((_EPT,), jnp.int32),
        pltpu.VMEM((_K, 128), f32),
        pltpu.VMEM((_K, 128), f32),
    ]
    if with_counts:
        scratch.append(pltpu.VMEM((_NR,), f32))
    scratch += [pltpu.SemaphoreType.DMA for _ in range(4)]
    mesh = plsc.VectorSubcoreMesh(core_axis_name="c", subcore_axis_name="s",
                                  num_cores=_NC, num_subcores=_NS)
    return pl.kernel(
        functools.partial(_sc_agg_body, n_pass, with_counts),
        out_type=out_type,
        mesh=mesh,
        scratch_types=scratch,
        compiler_params=pltpu.CompilerParams(needs_layout_passes=False),
    )


def _sc_agg1(*args):
    return _make_sc_agg(1, True)(*args)


def _sc_agg2(*args):
    return _make_sc_agg(2, False)(*args)


def _make_tc1(bn):
    f32 = jnp.float32

    def body(agg_r, cnt16_r, x_r, wl_r, wr_r, b_r, out_r, cnt_o):
        a = jnp.concatenate([agg_r[0], agg_r[1]], axis=1)
        xx = jnp.concatenate([x_r[0], x_r[1]], axis=1)
        cnt = jnp.sum(cnt16_r[...], axis=1)[:, None]
        cnt_o[...] = cnt
        inv = 1.0 / jnp.maximum(cnt, 1.0)
        h = jnp.dot(a * inv, wl_r[...], preferred_element_type=f32)
        h = h + jnp.dot(xx, wr_r[...], preferred_element_type=f32)
        h = jnp.maximum(h + b_r[...], 0.0)
        for j in range(4):
            out_r[j] = h[:, j * 128:(j + 1) * 128]

    return pl.pallas_call(
        body,
        grid=(_N // bn,),
        in_specs=[
            pl.BlockSpec((2, bn, 128), lambda i: (0, i, 0)),
            pl.BlockSpec((bn, _NS), lambda i: (i, 0)),
            pl.BlockSpec((2, bn, 128), lambda i: (0, i, 0)),
            pl.BlockSpec((256, 512), lambda i: (0, 0)),
            pl.BlockSpec((256, 512), lambda i: (0, 0)),
            pl.BlockSpec((1, 512), lambda i: (0, 0)),
        ],
        out_specs=[
            pl.BlockSpec((4, bn, 128), lambda i: (0, i, 0)),
            pl.BlockSpec((bn, 1), lambda i: (i, 0)),
        ],
        out_shape=[
            jax.ShapeDtypeStruct((4, _NP, 128), f32),
            jax.ShapeDtypeStruct((_N, 1), f32),
        ],
    )


def _make_tc2(bn):
    f32 = jnp.float32

    def body(agg_r, cnt_r, h_r, wl_r, wr_r, b_r, out_r):
        a = jnp.concatenate([agg_r[j] for j in range(4)], axis=1)
        hh = jnp.concatenate([h_r[j] for j in range(4)], axis=1)
        inv = 1.0 / jnp.maximum(cnt_r[...], 1.0)
        o = jnp.dot(a * inv, wl_r[...], preferred_element_type=f32)
        o = o + jnp.dot(hh, wr_r[...], preferred_element_type=f32)
        out_r[...] = o + b_r[...]

    return pl.pallas_call(
        body,
        grid=(_N // bn,),
        in_specs=[
            pl.BlockSpec((4, bn, 128), lambda i: (0, i, 0)),
            pl.BlockSpec((bn, 1), lambda i: (i, 0)),
            pl.BlockSpec((4, bn, 128), lambda i: (0, i, 0)),
            pl.BlockSpec((512, 512), lambda i: (0, 0)),
            pl.BlockSpec((512, 512), lambda i: (0, 0)),
            pl.BlockSpec((1, 512), lambda i: (0, 0)),
        ],
        out_specs=pl.BlockSpec((bn, 512), lambda i: (i, 0)),
        out_shape=jax.ShapeDtypeStruct((_N, 512), f32),
    )


_tc1 = _make_tc1(1000)
_tc2 = _make_tc2(1000)


def kernel(x, edge_index, Wl1, Wr1, b1, Wl2, Wr2, b2):
    f32 = jnp.float32
    src = edge_index[0].astype(jnp.int32)
    dst = edge_index[1].astype(jnp.int32)
    pad = _EP - _E
    fill = jnp.full((pad,), _PAD_ID, jnp.int32)
    src_r = jnp.concatenate([src, fill]).reshape(_NS, _EPT)
    dst_r = jnp.concatenate([dst, fill]).reshape(_NS, _EPT)

    xp = jnp.pad(x.astype(f32), ((0, _NP - _N), (0, 0)))
    x2 = xp.reshape(_NP, 2, 128).transpose(1, 0, 2)

    zrow = jnp.zeros((_OPT, 128), f32)
    zflat = jnp.zeros((_NR,), f32)

    agg1, cnt16 = _sc_agg1(x2, src_r, dst_r, zrow, zflat)
    h4, cntc = _tc1(agg1, cnt16.T, x2, Wl1.T, Wr1.T, b1.reshape(1, -1))
    (agg2,) = _sc_agg2(h4, src_r, dst_r, zrow)
    out = _tc2(agg2, cntc, h4, Wl2.T, Wr2.T, b2.reshape(1, -1))
    return out


# speedup vs baseline: 1.2591x; 1.2591x over previous
import functools

import jax
import jax.numpy as jnp
from jax import lax
from jax.experimental import pallas as pl
from jax.experimental.pallas import tpu as pltpu
from jax.experimental.pallas import tpu_sc as plsc

_N = 10000
_E = 160000
_NP = 10240
_NR = 10112
_NC = 2
_NS = 16
_K = 64
_EP = 163840
_EPT1 = _EP // (_NC * _NS)
_EPT2 = _EP // _NS
_OPT = _NR // _NS
_PAD_ID = 10000


def _edge_loop(table, acc, idx_s, idx_d, rows, gsem, ssem, dummy, ch, count):
    def g_issue(k, b):
        pltpu.async_copy(table.at[idx_s.at[pl.ds(k * _K, _K)]], rows[b],
                         gsem[b])

    def g_wait(b):
        pltpu.make_async_copy(dummy, rows[b], gsem[b]).wait()

    def s_issue(k, b):
        pltpu.async_copy(rows[b], acc.at[idx_d.at[pl.ds(k * _K, _K)]],
                         ssem[b], add=True)

    def s_wait(b):
        pltpu.make_async_copy(dummy, rows[b], ssem[b]).wait()

    g_issue(0, 0)
    g_wait(0)
    s_issue(0, 0)
    count(0)
    g_issue(1, 1)

    def steady(i, carry):
        for (d, b) in ((1, 1), (2, 0)):
            k = 2 * i + d
            g_wait(b)
            s_issue(k, b)
            count(k)
            s_wait(1 - b)
            g_issue(k + 1, 1 - b)
        return carry

    lax.fori_loop(0, (ch - 2) // 2, steady, 0)

    g_wait(1)
    s_issue(ch - 1, 1)
    count(ch - 1)
    s_wait(0)
    s_wait(1)


def _sc_l1_body(table, src_h, dst_h, zrow_h, zflat_h, agg_o, cnt_o,
                acc, idx_s, idx_d, rows0, rows1, hist,
                gsem0, gsem1, ssem0, ssem1):
    c = lax.axis_index("c")
    t = lax.axis_index("s")
    ch = _EPT1 // _K

    pltpu.sync_copy(src_h.at[c].at[t], idx_s)
    pltpu.sync_copy(dst_h.at[c].at[t], idx_d)
    pltpu.sync_copy(zflat_h, hist)
    pltpu.sync_copy(zrow_h, acc.at[pl.ds(t * _OPT, _OPT)])
    plsc.subcore_barrier()

    ones = jnp.full((16,), 1.0, jnp.float32)

    def count(k):
        for i in range(_K // 16):
            v = idx_d[pl.ds(k * _K + i * 16, 16)]
            plsc.addupdate_scatter(hist, [v], ones)

    dummy = zrow_h.at[pl.ds(0, _K)]
    _edge_loop(table, acc, idx_s, idx_d, (rows0, rows1),
               (gsem0, gsem1), (ssem0, ssem1), dummy, ch, count)
    plsc.subcore_barrier()

    pltpu.sync_copy(acc.at[pl.ds(t * _OPT, _OPT)],
                    agg_o.at[c].at[pl.ds(t * _OPT, _OPT)])
    pltpu.sync_copy(hist, cnt_o.at[c].at[t])


def _sc_l2_body(table, src_h, dst_h, zrow_h, agg_o,
                acc, idx_s, idx_d, rows0, rows1,
                gsem0, gsem1, ssem0, ssem1):
    c = lax.axis_index("c")
    t = lax.axis_index("s")
    ch = _EPT2 // _K

    pltpu.sync_copy(src_h.at[t], idx_s)
    pltpu.sync_copy(dst_h.at[t], idx_d)
    pltpu.sync_copy(zrow_h, acc.at[pl.ds(t * _OPT, _OPT)])
    plsc.subcore_barrier()

    dummy = zrow_h.at[pl.ds(0, _K)]
    _edge_loop(table.at[c], acc, idx_s, idx_d, (rows0, rows1),
               (gsem0, gsem1), (ssem0, ssem1), dummy, ch, lambda k: None)
    plsc.subcore_barrier()

    pltpu.sync_copy(acc.at[pl.ds(t * _OPT, _OPT)],
                    agg_o.at[c].at[pl.ds(t * _OPT, _OPT)])


@functools.lru_cache(maxsize=None)
def _make_sc_l1():
    bf16 = jnp.bfloat16
    mesh = plsc.VectorSubcoreMesh(core_axis_name="c", subcore_axis_name="s",
                                  num_cores=_NC, num_subcores=_NS)
    return pl.kernel(
        _sc_l1_body,
        out_type=[
            jax.ShapeDtypeStruct((_NC, _NR, 2, 128), bf16),
            jax.ShapeDtypeStruct((_NC, _NS, _NR), jnp.float32),
        ],
        mesh=mesh,
        scratch_types=[
            pltpu.VMEM_SHARED((_NR, 2, 128), bf16),
            pltpu.VMEM((_EPT1,), jnp.int32),
            pltpu.VMEM((_EPT1,), jnp.int32),
            pltpu.VMEM((_K, 2, 128), bf16),
            pltpu.VMEM((_K, 2, 128), bf16),
            pltpu.VMEM((_NR,), jnp.float32),
        ] + [pltpu.SemaphoreType.DMA] * 4,
        compiler_params=pltpu.CompilerParams(needs_layout_passes=False,
                                             use_tc_tiling_on_sc=False),
    )


@functools.lru_cache(maxsize=None)
def _make_sc_l2():
    bf16 = jnp.bfloat16
    mesh = plsc.VectorSubcoreMesh(core_axis_name="c", subcore_axis_name="s",
                                  num_cores=_NC, num_subcores=_NS)
    return pl.kernel(
        _sc_l2_body,
        out_type=[jax.ShapeDtypeStruct((_NC, _NR, 2, 128), bf16)],
        mesh=mesh,
        scratch_types=[
            pltpu.VMEM_SHARED((_NR, 2, 128), bf16),
            pltpu.VMEM((_EPT2,), jnp.int32),
            pltpu.VMEM((_EPT2,), jnp.int32),
            pltpu.VMEM((_K, 2, 128), bf16),
            pltpu.VMEM((_K, 2, 128), bf16),
        ] + [pltpu.SemaphoreType.DMA] * 4,
        compiler_params=pltpu.CompilerParams(needs_layout_passes=False,
                                             use_tc_tiling_on_sc=False),
    )


def _sc_agg1(*args):
    return _make_sc_l1()(*args)


def _sc_agg2(*args):
    return _make_sc_l2()(*args)


def _make_tc1(bn):
    f32 = jnp.float32

    def body(agg_r, cnt_r, x_r, wl_r, wr_r, b_r, out_r, cnt_o):
        ap = agg_r[0].astype(f32) + agg_r[1].astype(f32)
        a = jnp.concatenate([ap[:, 0, :], ap[:, 1, :]], axis=1)
        cnt = jnp.sum(cnt_r[...], axis=1)[:, None]
        cnt_o[...] = cnt
        inv = 1.0 / jnp.maximum(cnt, 1.0)
        h = jnp.dot(a * inv, wl_r[...], preferred_element_type=f32)
        h = h + jnp.dot(x_r[...], wr_r[...], preferred_element_type=f32)
        h = jnp.maximum(h + b_r[...], 0.0).astype(jnp.bfloat16)
        for j in range(2):
            for s in range(2):
                out_r[j, :, s, :] = h[:, j * 256 + s * 128:
                                      j * 256 + (s + 1) * 128]

    return pl.pallas_call(
        body,
        grid=(_N // bn,),
        in_specs=[
            pl.BlockSpec((2, bn, 2, 128), lambda i: (0, i, 0, 0)),
            pl.BlockSpec((bn, 32), lambda i: (i, 0)),
            pl.BlockSpec((bn, 256), lambda i: (i, 0)),
            pl.BlockSpec((256, 512), lambda i: (0, 0)),
            pl.BlockSpec((256, 512), lambda i: (0, 0)),
            pl.BlockSpec((1, 512), lambda i: (0, 0)),
        ],
        out_specs=[
            pl.BlockSpec((2, bn, 2, 128), lambda i: (0, i, 0, 0)),
            pl.BlockSpec((bn, 1), lambda i: (i, 0)),
        ],
        out_shape=[
            jax.ShapeDtypeStruct((2, _NP, 2, 128), jnp.bfloat16),
            jax.ShapeDtypeStruct((_N, 1), f32),
        ],
    )


def _make_tc2(bn):
    f32 = jnp.float32

    def body(agg_r, cnt_r, h_r, wl_r, wr_r, b_r, out_r):
        a = jnp.concatenate([agg_r[j][:, s, :].astype(f32)
                             for j in range(2) for s in range(2)], axis=1)
        hh = jnp.concatenate([h_r[j][:, s, :].astype(f32)
                              for j in range(2) for s in range(2)], axis=1)
        inv = 1.0 / jnp.maximum(cnt_r[...], 1.0)
        o = jnp.dot(a * inv, wl_r[...], preferred_element_type=f32)
        o = o + jnp.dot(hh, wr_r[...], preferred_element_type=f32)
        out_r[...] = o + b_r[...]

    return pl.pallas_call(
        body,
        grid=(_N // bn,),
        in_specs=[
            pl.BlockSpec((2, bn, 2, 128), lambda i: (0, i, 0, 0)),
            pl.BlockSpec((bn, 1), lambda i: (i, 0)),
            pl.BlockSpec((2, bn, 2, 128), lambda i: (0, i, 0, 0)),
            pl.BlockSpec((512, 512), lambda i: (0, 0)),
            pl.BlockSpec((512, 512), lambda i: (0, 0)),
            pl.BlockSpec((1, 512), lambda i: (0, 0)),
        ],
        out_specs=pl.BlockSpec((bn, 512), lambda i: (i, 0)),
        out_shape=jax.ShapeDtypeStruct((_N, 512), f32),
    )


_tc1 = _make_tc1(1000)
_tc2 = _make_tc2(1000)


def kernel(x, edge_index, Wl1, Wr1, b1, Wl2, Wr2, b2):
    f32 = jnp.float32
    bf16 = jnp.bfloat16
    src = edge_index[0].astype(jnp.int32)
    dst = edge_index[1].astype(jnp.int32)
    fill = jnp.full((_EP - _E,), _PAD_ID, jnp.int32)
    src_p = jnp.concatenate([src, fill])
    dst_p = jnp.concatenate([dst, fill])
    src1 = src_p.reshape(_NC, _NS, _EPT1)
    dst1 = dst_p.reshape(_NC, _NS, _EPT1)
    src2 = src_p.reshape(_NS, _EPT2)
    dst2 = dst_p.reshape(_NS, _EPT2)

    xp = jnp.pad(x.astype(f32), ((0, _NP - _N), (0, 0)))
    xb = xp.astype(bf16).reshape(_NP, 2, 128)

    zrow = jnp.zeros((_OPT, 2, 128), bf16)
    zflat = jnp.zeros((_NR,), f32)

    agg1, cnt32 = _sc_agg1(xb, src1, dst1, zrow, zflat)
    cnt_t = cnt32.reshape(_NC * _NS, _NR).T
    h2, cntc = _tc1(agg1, cnt_t, xp, Wl1.T, Wr1.T, b1.reshape(1, -1))
    (agg2,) = _sc_agg2(h2, src2, dst2, zrow)
    out = _tc2(agg2, cntc, h2, Wl2.T, Wr2.T, b2.reshape(1, -1))
    return out


# speedup vs baseline: 1.2940x; 1.0277x over previous
import functools

import jax
import jax.numpy as jnp
from jax import lax
from jax.experimental import pallas as pl
from jax.experimental.pallas import tpu as pltpu
from jax.experimental.pallas import tpu_sc as plsc

_N = 10000
_E = 160000
_NP = 10240
_NR = 10112
_NC = 2
_NS = 16
_K = 64
_EP = 163840
_EPT1 = _EP // (_NC * _NS)
_EPT2 = _EP // _NS
_OPT = _NR // _NS
_PAD_ID = 10000


def _edge_loop(table, acc, idx_s, idx_d, rows, gsem, ssem, dummy, ch, count):
    def g_issue(k, b):
        pltpu.async_copy(table.at[idx_s.at[pl.ds(k * _K, _K)]], rows[b],
                         gsem[b])

    def g_wait(b):
        pltpu.make_async_copy(dummy, rows[b], gsem[b]).wait()

    def s_issue(k, b):
        pltpu.async_copy(rows[b], acc.at[idx_d.at[pl.ds(k * _K, _K)]],
                         ssem[b], add=True)

    def s_wait(b):
        pltpu.make_async_copy(dummy, rows[b], ssem[b]).wait()

    g_issue(0, 0)
    g_wait(0)
    s_issue(0, 0)
    count(0)
    g_issue(1, 1)

    def steady(i, carry):
        for (d, b) in ((1, 1), (2, 0)):
            k = 2 * i + d
            g_wait(b)
            s_issue(k, b)
            count(k)
            s_wait(1 - b)
            g_issue(k + 1, 1 - b)
        return carry

    lax.fori_loop(0, (ch - 2) // 2, steady, 0)

    g_wait(1)
    s_issue(ch - 1, 1)
    count(ch - 1)
    s_wait(0)
    s_wait(1)


def _sc_l1_body(table, src_h, dst_h, zrow_h, zflat_h, agg_o, cnt_o,
                acc, idx_s, idx_d, rows0, rows1, hist,
                gsem0, gsem1, ssem0, ssem1):
    c = lax.axis_index("c")
    t = lax.axis_index("s")
    ch = _EPT1 // _K

    pltpu.sync_copy(src_h.at[c].at[t], idx_s)
    pltpu.sync_copy(dst_h.at[c].at[t], idx_d)
    pltpu.sync_copy(zflat_h, hist)
    pltpu.sync_copy(zrow_h, acc.at[pl.ds(t * _OPT, _OPT)])
    plsc.subcore_barrier()

    ones = jnp.full((16,), 1.0, jnp.float32)

    def count(k):
        for i in range(_K // 16):
            v = idx_d[pl.ds(k * _K + i * 16, 16)]
            plsc.addupdate_scatter(hist, [v], ones)

    dummy = zrow_h.at[pl.ds(0, _K)]
    _edge_loop(table, acc, idx_s, idx_d, (rows0, rows1),
               (gsem0, gsem1), (ssem0, ssem1), dummy, ch, count)
    plsc.subcore_barrier()

    pltpu.sync_copy(acc.at[pl.ds(t * _OPT, _OPT)],
                    agg_o.at[c].at[pl.ds(t * _OPT, _OPT)])
    pltpu.sync_copy(hist, cnt_o.at[c].at[t])


def _sc_l2_body(table, src_h, dst_h, zrow_h, agg_o,
                acc, idx_s, idx_d, rows0, rows1,
                gsem0, gsem1, ssem0, ssem1):
    c = lax.axis_index("c")
    t = lax.axis_index("s")
    ch = _EPT2 // _K

    pltpu.sync_copy(src_h.at[t], idx_s)
    pltpu.sync_copy(dst_h.at[t], idx_d)
    pltpu.sync_copy(zrow_h, acc.at[pl.ds(t * _OPT, _OPT)])
    plsc.subcore_barrier()

    dummy = zrow_h.at[pl.ds(0, _K)]
    _edge_loop(table.at[c], acc, idx_s, idx_d, (rows0, rows1),
               (gsem0, gsem1), (ssem0, ssem1), dummy, ch, lambda k: None)
    plsc.subcore_barrier()

    pltpu.sync_copy(acc.at[pl.ds(t * _OPT, _OPT)],
                    agg_o.at[c].at[pl.ds(t * _OPT, _OPT)])


@functools.lru_cache(maxsize=None)
def _make_sc_l1():
    bf16 = jnp.bfloat16
    mesh = plsc.VectorSubcoreMesh(core_axis_name="c", subcore_axis_name="s",
                                  num_cores=_NC, num_subcores=_NS)
    return pl.kernel(
        _sc_l1_body,
        out_type=[
            jax.ShapeDtypeStruct((_NC, _NR, 2, 128), bf16),
            jax.ShapeDtypeStruct((_NC, _NS, _NR), jnp.float32),
        ],
        mesh=mesh,
        scratch_types=[
            pltpu.VMEM_SHARED((_NR, 2, 128), bf16),
            pltpu.VMEM((_EPT1,), jnp.int32),
            pltpu.VMEM((_EPT1,), jnp.int32),
            pltpu.VMEM((_K, 2, 128), bf16),
            pltpu.VMEM((_K, 2, 128), bf16),
            pltpu.VMEM((_NR,), jnp.float32),
        ] + [pltpu.SemaphoreType.DMA] * 4,
        compiler_params=pltpu.CompilerParams(needs_layout_passes=False,
                                             use_tc_tiling_on_sc=False),
    )


@functools.lru_cache(maxsize=None)
def _make_sc_l2():
    bf16 = jnp.bfloat16
    mesh = plsc.VectorSubcoreMesh(core_axis_name="c", subcore_axis_name="s",
                                  num_cores=_NC, num_subcores=_NS)
    return pl.kernel(
        _sc_l2_body,
        out_type=[jax.ShapeDtypeStruct((_NC, _NR, 2, 128), bf16)],
        mesh=mesh,
        scratch_types=[
            pltpu.VMEM_SHARED((_NR, 2, 128), bf16),
            pltpu.VMEM((_EPT2,), jnp.int32),
            pltpu.VMEM((_EPT2,), jnp.int32),
            pltpu.VMEM((_K, 2, 128), bf16),
            pltpu.VMEM((_K, 2, 128), bf16),
        ] + [pltpu.SemaphoreType.DMA] * 4,
        compiler_params=pltpu.CompilerParams(needs_layout_passes=False,
                                             use_tc_tiling_on_sc=False),
    )


def _sc_agg1(*args):
    return _make_sc_l1()(*args)


def _sc_agg2(*args):
    return _make_sc_l2()(*args)


def _make_tc_r1(bn):
    f32 = jnp.float32

    def body(x_r, wr_r, b_r, out_r):
        xx = jnp.concatenate([x_r[:, 0, :], x_r[:, 1, :]], axis=1)
        out_r[...] = jnp.dot(xx, wr_r[...],
                             preferred_element_type=f32) + b_r[...]

    return pl.pallas_call(
        body,
        grid=(_N // bn,),
        in_specs=[
            pl.BlockSpec((bn, 2, 128), lambda i: (i, 0, 0)),
            pl.BlockSpec((256, 512), lambda i: (0, 0)),
            pl.BlockSpec((1, 512), lambda i: (0, 0)),
        ],
        out_specs=pl.BlockSpec((bn, 512), lambda i: (i, 0)),
        out_shape=jax.ShapeDtypeStruct((_N, 512), f32),
    )


def _make_tc1b(bn):
    f32 = jnp.float32

    def body(agg_r, cnt_r, r1_r, wl_r, out_r, cnt_o):
        ap = agg_r[0].astype(f32) + agg_r[1].astype(f32)
        a = jnp.concatenate([ap[:, 0, :], ap[:, 1, :]], axis=1)
        cnt = jnp.sum(cnt_r[...], axis=1)[:, None]
        cnt_o[...] = cnt
        inv = 1.0 / jnp.maximum(cnt, 1.0)
        ab = (a * inv).astype(jnp.bfloat16)
        h = jnp.dot(ab, wl_r[...], preferred_element_type=f32) + r1_r[...]
        h = jnp.maximum(h, 0.0).astype(jnp.bfloat16)
        for j in range(2):
            for t in range(2):
                out_r[j, :, t, :] = h[:, j * 256 + t * 128:
                                      j * 256 + (t + 1) * 128]

    return pl.pallas_call(
        body,
        grid=(_N // bn,),
        in_specs=[
            pl.BlockSpec((2, bn, 2, 128), lambda i: (0, i, 0, 0)),
            pl.BlockSpec((bn, 32), lambda i: (i, 0)),
            pl.BlockSpec((bn, 512), lambda i: (i, 0)),
            pl.BlockSpec((256, 512), lambda i: (0, 0)),
        ],
        out_specs=[
            pl.BlockSpec((2, bn, 2, 128), lambda i: (0, i, 0, 0)),
            pl.BlockSpec((bn, 1), lambda i: (i, 0)),
        ],
        out_shape=[
            jax.ShapeDtypeStruct((2, _NP, 2, 128), jnp.bfloat16),
            jax.ShapeDtypeStruct((_N, 1), f32),
        ],
    )


def _make_tc2a(bn):
    f32 = jnp.float32

    def body(h_r, wr_r, out_r):
        hh = jnp.concatenate([h_r[j][:, t, :]
                              for j in range(2) for t in range(2)], axis=1)
        out_r[...] = jnp.dot(hh, wr_r[...], preferred_element_type=f32)

    return pl.pallas_call(
        body,
        grid=(_N // bn,),
        in_specs=[
            pl.BlockSpec((2, bn, 2, 128), lambda i: (0, i, 0, 0)),
            pl.BlockSpec((512, 512), lambda i: (0, 0)),
        ],
        out_specs=pl.BlockSpec((bn, 512), lambda i: (i, 0)),
        out_shape=jax.ShapeDtypeStruct((_N, 512), f32),
    )


def _make_tc2b(bn):
    f32 = jnp.float32

    def body(agg_r, cnt_r, r2_r, wl_r, b_r, out_r):
        a = jnp.concatenate([agg_r[j][:, t, :].astype(f32)
                             for j in range(2) for t in range(2)], axis=1)
        inv = 1.0 / jnp.maximum(cnt_r[...], 1.0)
        ab = (a * inv).astype(jnp.bfloat16)
        o = jnp.dot(ab, wl_r[...], preferred_element_type=f32)
        out_r[...] = o + r2_r[...] + b_r[...]

    return pl.pallas_call(
        body,
        grid=(_N // bn,),
        in_specs=[
            pl.BlockSpec((2, bn, 2, 128), lambda i: (0, i, 0, 0)),
            pl.BlockSpec((bn, 1), lambda i: (i, 0)),
            pl.BlockSpec((bn, 512), lambda i: (i, 0)),
            pl.BlockSpec((512, 512), lambda i: (0, 0)),
            pl.BlockSpec((1, 512), lambda i: (0, 0)),
        ],
        out_specs=pl.BlockSpec((bn, 512), lambda i: (i, 0)),
        out_shape=jax.ShapeDtypeStruct((_N, 512), f32),
    )


_tc_r1 = _make_tc_r1(1000)
_tc1b = _make_tc1b(1000)
_tc2a = _make_tc2a(1000)
_tc2b = _make_tc2b(1000)


def kernel(x, edge_index, Wl1, Wr1, b1, Wl2, Wr2, b2):
    f32 = jnp.float32
    bf16 = jnp.bfloat16
    src = edge_index[0].astype(jnp.int32)
    dst = edge_index[1].astype(jnp.int32)
    fill = jnp.full((_EP - _E,), _PAD_ID, jnp.int32)
    src_p = jnp.concatenate([src, fill])
    dst_p = jnp.concatenate([dst, fill])
    src1 = src_p.reshape(_NC, _NS, _EPT1)
    dst1 = dst_p.reshape(_NC, _NS, _EPT1)
    src2 = src_p.reshape(_NS, _EPT2)
    dst2 = dst_p.reshape(_NS, _EPT2)

    xp = jnp.pad(x.astype(f32), ((0, _NP - _N), (0, 0)))
    xb = xp.astype(bf16).reshape(_NP, 2, 128)

    zrow = jnp.zeros((_OPT, 2, 128), bf16)
    zflat = jnp.zeros((_NR,), f32)

    r1 = _tc_r1(xb, Wr1.T.astype(bf16), b1.reshape(1, -1))
    agg1, cnt32 = _sc_agg1(xb, src1, dst1, zrow, zflat)
    cnt_t = cnt32.reshape(_NC * _NS, _NR).T
    h2, cntc = _tc1b(agg1, cnt_t, r1, Wl1.T.astype(bf16))
    r2 = _tc2a(h2, Wr2.T.astype(bf16))
    (agg2,) = _sc_agg2(h2, src2, dst2, zrow)
    out = _tc2b(agg2, cntc, r2, Wl2.T.astype(bf16), b2.reshape(1, -1))
    return out
